# baseline (speedup 1.0000x reference)
import jax
import jax.numpy as jnp
from jax import lax
from jax.experimental import pallas as pl
from jax.experimental.pallas import tpu as pltpu
from jax.experimental.pallas import tpu_sc as plsc

B, T, F = 8, 2048, 128
K = 1408
R = B * T
L = 16
NC, NS = 2, 16
NW = NC * NS
RPW = R // NW
BLK = 32
NBLK = RPW // BLK
NCHUNK = K // L
G = 8
NGROUP = NCHUNK // G


def _sc_gather(x_hbm, idx_hbm, out_hbm, idx_v, xin0, xin1, xout0, xout1,
               isem0, isem1, osem0, osem1):
    wid = lax.axis_index("s") * NC + lax.axis_index("c")
    pltpu.sync_copy(idx_hbm, idx_v)

    def in_copy(b, buf, sem):
        base = (wid * NBLK + b) * BLK
        return pltpu.make_async_copy(x_hbm.at[pl.ds(base, BLK)], buf, sem)

    def out_copy(b, buf, sem):
        base = (wid * NBLK + b) * BLK
        return pltpu.make_async_copy(buf, out_hbm.at[pl.ds(base, BLK)], sem)

    zero = jnp.zeros((L,), jnp.int32)
    HB = BLK // 2

    def out_copy_half(b, h, buf, sem):
        base = (wid * NBLK + b) * BLK + h * HB
        return pltpu.make_async_copy(
            buf.at[pl.ds(h * HB, HB)], out_hbm.at[pl.ds(base, HB)], sem)

    def compute(xin, xout, r0):
        for g0, ng in ((0, 4), (4, 4), (8, 3)):
            cb = g0 * G * L
            nk = ng * G
            idxs = [idx_v[pl.ds(cb + k * L, L)] for k in range(nk)]

            @plsc.parallel_loop(r0, r0 + HB, unroll=1)
            def _row(r):
                rb = r * F
                for k in range(nk):
                    vals = plsc.load_gather(xin, [zero, idxs[k] + rb])
                    xout[r, pl.ds(cb + k * L, L)] = vals

    in_copy(0, xin0, isem0).start()
    in_copy(1, xin1, isem1).start()

    @pl.loop(0, NBLK // 2)
    def _pair(m):
        for p, (xin, xout, isem, osem) in enumerate(
                ((xin0, xout0, isem0, osem0), (xin1, xout1, isem1, osem1))):
            b = 2 * m + p
            in_copy(b, xin, isem).wait()

            @pl.when(m > 0)
            def _():
                out_copy_half(b - 2, 0, xout, osem).wait()
                out_copy_half(b - 2, 1, xout, osem).wait()

            compute(xin, xout, 0)
            out_copy_half(b, 0, xout, osem).start()
            compute(xin, xout, HB)
            out_copy_half(b, 1, xout, osem).start()

            @pl.when(m < NBLK // 2 - 1)
            def _():
                in_copy(b + 2, xin, isem).start()

    for b, (xout, osem) in ((NBLK - 2, (xout0, osem0)),
                            (NBLK - 1, (xout1, osem1))):
        out_copy_half(b, 0, xout, osem).wait()
        out_copy_half(b, 1, xout, osem).wait()


@jax.jit
def _run(x, idx):
    mesh = plsc.VectorSubcoreMesh(
        core_axis_name="c", subcore_axis_name="s",
        num_cores=NC, num_subcores=NS)
    f = pl.kernel(
        _sc_gather,
        out_type=jax.ShapeDtypeStruct((R, K), jnp.float32),
        mesh=mesh,
        compiler_params=pltpu.CompilerParams(needs_layout_passes=False),
        scratch_types=[
            pltpu.VMEM((K,), jnp.int32),
            pltpu.VMEM((BLK, F), jnp.float32),
            pltpu.VMEM((BLK, F), jnp.float32),
            pltpu.VMEM((BLK, K), jnp.float32),
            pltpu.VMEM((BLK, K), jnp.float32),
            pltpu.SemaphoreType.DMA,
            pltpu.SemaphoreType.DMA,
            pltpu.SemaphoreType.DMA,
            pltpu.SemaphoreType.DMA,
        ],
    )
    return f(x, idx)


def kernel(inputs, kernel):
    x = inputs.reshape(R, F)
    out = _run(x, kernel)
    return out.reshape(B, T, K)


# speedup vs baseline: 1.4427x; 1.4427x over previous
import jax
import jax.numpy as jnp
from jax import lax
from jax.experimental import pallas as pl
from jax.experimental.pallas import tpu as pltpu
from jax.experimental.pallas import tpu_sc as plsc

B, T, F = 8, 2048, 128
K = 1408
R = B * T
L = 16
NC, NS = 2, 16
NW = NC * NS
RPW = R // NW
BLK = 32
NBLK = RPW // BLK
NCHUNK = K // L
G = 8
NGROUP = NCHUNK // G


def _sc_gather(x_hbm, idx_hbm, out_hbm, idx_v, xin0, xin1, xout0, xout1,
               isem0, isem1, osem0, osem1):
    wid = lax.axis_index("s") * NC + lax.axis_index("c")
    pltpu.sync_copy(idx_hbm, idx_v)

    def in_copy(b, buf, sem):
        base = (wid * NBLK + b) * BLK
        return pltpu.make_async_copy(x_hbm.at[pl.ds(base, BLK)], buf, sem)

    def out_copy(b, buf, sem):
        base = (wid * NBLK + b) * BLK
        return pltpu.make_async_copy(buf, out_hbm.at[pl.ds(base, BLK)], sem)

    zero = jnp.zeros((L,), jnp.int32)

    def compute(xin, xout):
        for g0, ng in ((0, 4), (4, 4), (8, 3)):
            cb = g0 * G * L
            nk = ng * G
            idxs = [idx_v[pl.ds(cb + k * L, L)] for k in range(nk)]

            @plsc.parallel_loop(0, BLK, unroll=1)
            def _row(r):
                rb = r * F
                for k in range(nk):
                    vals = plsc.load_gather(xin, [zero, idxs[k] + rb])
                    xout[r, pl.ds(cb + k * L, L)] = vals

    in_copy(0, xin0, isem0).start()
    in_copy(1, xin1, isem1).start()

    @pl.loop(0, NBLK // 2)
    def _pair(m):
        for p, (xin, xout, isem, osem) in enumerate(
                ((xin0, xout0, isem0, osem0), (xin1, xout1, isem1, osem1))):
            b = 2 * m + p
            in_copy(b, xin, isem).wait()

            @pl.when(m > 0)
            def _():
                out_copy(b - 2, xout, osem).wait()

            compute(xin, xout)
            out_copy(b, xout, osem).start()

            @pl.when(m < NBLK // 2 - 1)
            def _():
                in_copy(b + 2, xin, isem).start()

    out_copy(NBLK - 2, xout0, osem0).wait()
    out_copy(NBLK - 1, xout1, osem1).wait()


@jax.jit
def _run(x, idx):
    mesh = plsc.VectorSubcoreMesh(
        core_axis_name="c", subcore_axis_name="s",
        num_cores=NC, num_subcores=NS)
    f = pl.kernel(
        _sc_gather,
        out_type=jax.ShapeDtypeStruct((R, K), jnp.float32),
        mesh=mesh,
        compiler_params=pltpu.CompilerParams(needs_layout_passes=False),
        scratch_types=[
            pltpu.VMEM((K,), jnp.int32),
            pltpu.VMEM((BLK, F), jnp.float32),
            pltpu.VMEM((BLK, F), jnp.float32),
            pltpu.VMEM((BLK, K), jnp.float32),
            pltpu.VMEM((BLK, K), jnp.float32),
            pltpu.SemaphoreType.DMA,
            pltpu.SemaphoreType.DMA,
            pltpu.SemaphoreType.DMA,
            pltpu.SemaphoreType.DMA,
        ],
    )
    return f(x, idx)


def kernel(inputs, kernel):
    x = inputs.reshape(R, F)
    out = _run(x, kernel)
    return out.reshape(B, T, K)


# speedup vs baseline: 1.4542x; 1.0080x over previous
import jax
import jax.numpy as jnp
from jax import lax
from jax.experimental import pallas as pl
from jax.experimental.pallas import tpu as pltpu
from jax.experimental.pallas import tpu_sc as plsc

B, T, F = 8, 2048, 128
K = 1408
R = B * T
L = 16
NC, NS = 2, 16
NW = NC * NS
RPW = R // NW
BLK = 32
NBLK = RPW // BLK
NCHUNK = K // L
G = 8
NGROUP = NCHUNK // G


def _sc_gather(x_hbm, idx_hbm, out_hbm, idx_v, xin0, xin1, xout0, xout1,
               isem0, isem1, osem0, osem1):
    wid = lax.axis_index("s") * NC + lax.axis_index("c")

    def in_copy(b, buf, sem):
        base = (wid * NBLK + b) * BLK
        return pltpu.make_async_copy(x_hbm.at[pl.ds(base, BLK)], buf, sem)

    def out_copy(b, buf, sem):
        base = (wid * NBLK + b) * BLK
        return pltpu.make_async_copy(buf, out_hbm.at[pl.ds(base, BLK)], sem)

    zero = jnp.zeros((L,), jnp.int32)

    def compute(xin, xout):
        for g0, ng in ((0, 4), (4, 4), (8, 3)):
            cb = g0 * G * L
            nk = ng * G
            idxs = [idx_v[pl.ds(cb + k * L, L)] for k in range(nk)]

            @plsc.parallel_loop(0, BLK, unroll=1)
            def _row(r):
                rb = r * F
                for k in range(nk):
                    vals = plsc.load_gather(xin, [zero, idxs[k] + rb])
                    xout[r, pl.ds(cb + k * L, L)] = vals

    in_copy(0, xin0, isem0).start()
    in_copy(1, xin1, isem1).start()
    pltpu.sync_copy(idx_hbm, idx_v)

    @pl.loop(0, NBLK // 2)
    def _pair(m):
        for p, (xin, xout, isem, osem) in enumerate(
                ((xin0, xout0, isem0, osem0), (xin1, xout1, isem1, osem1))):
            b = 2 * m + p
            in_copy(b, xin, isem).wait()

            @pl.when(m > 0)
            def _():
                out_copy(b - 2, xout, osem).wait()

            compute(xin, xout)
            out_copy(b, xout, osem).start()

            @pl.when(m < NBLK // 2 - 1)
            def _():
                in_copy(b + 2, xin, isem).start()

    out_copy(NBLK - 2, xout0, osem0).wait()
    out_copy(NBLK - 1, xout1, osem1).wait()


@jax.jit
def _run(x, idx):
    mesh = plsc.VectorSubcoreMesh(
        core_axis_name="c", subcore_axis_name="s",
        num_cores=NC, num_subcores=NS)
    f = pl.kernel(
        _sc_gather,
        out_type=jax.ShapeDtypeStruct((R, K), jnp.float32),
        mesh=mesh,
        compiler_params=pltpu.CompilerParams(needs_layout_passes=False),
        scratch_types=[
            pltpu.VMEM((K,), jnp.int32),
            pltpu.VMEM((BLK, F), jnp.float32),
            pltpu.VMEM((BLK, F), jnp.float32),
            pltpu.VMEM((BLK, K), jnp.float32),
            pltpu.VMEM((BLK, K), jnp.float32),
            pltpu.SemaphoreType.DMA,
            pltpu.SemaphoreType.DMA,
            pltpu.SemaphoreType.DMA,
            pltpu.SemaphoreType.DMA,
        ],
    )
    return f(x, idx)


def kernel(inputs, kernel):
    x = inputs.reshape(R, F)
    out = _run(x, kernel)
    return out.reshape(B, T, K)
